# baseline (speedup 1.0000x reference)
import jax
import jax.numpy as jnp
from jax.experimental import pallas as pl
from jax.experimental.pallas import tpu as pltpu

_B, _C, _H, _W = 16, 64, 32, 32
_HW = _H * _W
_K = 512
_NS = 2
_S = _HW // _NS


def _vq_body(z_ref, zsq_ref, cb_ref, zq_ref, idx_ref):
    zb = z_ref[0]
    cb = cb_ref[...]
    dot = jax.lax.dot_general(
        cb, zb, (((1,), (0,)), ((), ())),
        preferred_element_type=jnp.float32,
    )
    zsq = zsq_ref[0]
    esq = jnp.sum(cb * cb, axis=1, keepdims=True)
    d = zsq - 2.0 * dot + esq
    dmin = jnp.min(d, axis=0, keepdims=True)
    kio = jax.lax.broadcasted_iota(jnp.int32, (_K, _S), 0)
    idx = jnp.min(jnp.where(d == dmin, kio, _K), axis=0).astype(jnp.int32)
    oh = (kio == idx[None, :]).astype(jnp.float32)
    zq = jax.lax.dot_general(
        cb, oh, (((0,), (0,)), ((), ())),
        preferred_element_type=jnp.float32,
        precision=jax.lax.Precision.HIGHEST,
    )
    zq_ref[0] = zq
    idx_ref[0, 0] = idx.reshape(_S // 128, 128)


def kernel(z_e, codebook):
    B, C, H, W = z_e.shape
    z = z_e.reshape(B, C, _HW)
    zsq = jnp.sum(
        jnp.transpose(z_e, (0, 2, 3, 1)).reshape(-1, C) ** 2, axis=1
    ).reshape(B, 1, _HW)
    zq, idx = pl.pallas_call(
        _vq_body,
        grid=(B, _NS),
        in_specs=[
            pl.BlockSpec((1, C, _S), lambda b, s: (b, 0, s)),
            pl.BlockSpec((1, 1, _S), lambda b, s: (b, 0, s)),
            pl.BlockSpec((_K, C), lambda b, s: (0, 0)),
        ],
        out_specs=[
            pl.BlockSpec((1, C, _S), lambda b, s: (b, 0, s)),
            pl.BlockSpec((1, 1, _S // 128, 128), lambda b, s: (b, s, 0, 0)),
        ],
        out_shape=[
            jax.ShapeDtypeStruct((B, C, _HW), jnp.float32),
            jax.ShapeDtypeStruct((B, _NS, _S // 128, 128), jnp.int32),
        ],
        compiler_params=pltpu.CompilerParams(
            dimension_semantics=("arbitrary", "arbitrary"),
        ),
    )(z, zsq, codebook)
    return zq.reshape(B, C, H, W), idx.reshape(-1)


# speedup vs baseline: 1.5615x; 1.5615x over previous
import jax
import jax.numpy as jnp
from jax.experimental import pallas as pl
from jax.experimental.pallas import tpu as pltpu

_B, _C, _H, _W = 16, 64, 32, 32
_HW = _H * _W
_K = 512


def _vq_body(z_ref, zsq_ref, cb_ref, zq_ref, idx_ref):
    zb = z_ref[0]
    cb = cb_ref[...]
    dot = jax.lax.dot_general(
        cb, zb, (((1,), (0,)), ((), ())),
        preferred_element_type=jnp.float32,
    )
    zsq = zsq_ref[0]
    esq = jnp.sum(cb * cb, axis=1, keepdims=True)
    d = zsq - 2.0 * dot + esq
    dmin = jnp.min(d, axis=0, keepdims=True)
    kio = jax.lax.broadcasted_iota(jnp.int32, (_K, _HW), 0)
    idx = jnp.min(jnp.where(d == dmin, kio, _K), axis=0).astype(jnp.int32)
    oh = (kio == idx[None, :]).astype(jnp.float32)
    zq = jax.lax.dot_general(
        cb, oh, (((0,), (0,)), ((), ())),
        preferred_element_type=jnp.float32,
    )
    zq_ref[0] = zq
    idx_ref[0] = idx.reshape(8, 128)


def kernel(z_e, codebook):
    B, C, H, W = z_e.shape
    z = z_e.reshape(B, C, H * W)
    zsq = jnp.sum(
        jnp.transpose(z_e, (0, 2, 3, 1)).reshape(-1, C) ** 2, axis=1
    ).reshape(B, 1, H * W)
    zq, idx = pl.pallas_call(
        _vq_body,
        grid=(B,),
        in_specs=[
            pl.BlockSpec((1, C, H * W), lambda b: (b, 0, 0)),
            pl.BlockSpec((1, 1, H * W), lambda b: (b, 0, 0)),
            pl.BlockSpec((_K, C), lambda b: (0, 0)),
        ],
        out_specs=[
            pl.BlockSpec((1, C, H * W), lambda b: (b, 0, 0)),
            pl.BlockSpec((1, 8, 128), lambda b: (b, 0, 0)),
        ],
        out_shape=[
            jax.ShapeDtypeStruct((B, C, H * W), jnp.float32),
            jax.ShapeDtypeStruct((B, 8, 128), jnp.int32),
        ],
        compiler_params=pltpu.CompilerParams(
            dimension_semantics=("arbitrary",),
        ),
    )(z, zsq, codebook)
    return zq.reshape(B, C, H, W), idx.reshape(-1)


# speedup vs baseline: 1.8441x; 1.1810x over previous
import jax
import jax.numpy as jnp
from jax.experimental import pallas as pl
from jax.experimental.pallas import tpu as pltpu

_B, _C, _H, _W = 16, 64, 32, 32
_HW = _H * _W
_K = 512


def _vq_body(z_ref, cb_ref, zq_ref, idx_ref):
    zb = z_ref[0]
    cb = cb_ref[...]
    dot = jax.lax.dot_general(
        cb, zb, (((1,), (0,)), ((), ())),
        preferred_element_type=jnp.float32,
    )
    s = zb * zb
    w = _C
    while w > 1:
        w //= 2
        s = s[:w] + s[w:2 * w]
    zsq = s
    esq = jnp.sum(cb * cb, axis=1, keepdims=True)
    d = zsq - 2.0 * dot + esq
    dmin = jnp.min(d, axis=0, keepdims=True)
    kio = jax.lax.broadcasted_iota(jnp.int32, (_K, _HW), 0)
    idx = jnp.min(jnp.where(d == dmin, kio, _K), axis=0).astype(jnp.int32)
    oh = (kio == idx[None, :]).astype(jnp.float32)
    zq = jax.lax.dot_general(
        cb, oh, (((0,), (0,)), ((), ())),
        preferred_element_type=jnp.float32,
    )
    zq_ref[0] = zq
    idx_ref[0] = idx.reshape(8, 128)


def kernel(z_e, codebook):
    B, C, H, W = z_e.shape
    z = z_e.reshape(B, C, H * W)
    zq, idx = pl.pallas_call(
        _vq_body,
        grid=(B,),
        in_specs=[
            pl.BlockSpec((1, C, H * W), lambda b: (b, 0, 0)),
            pl.BlockSpec((_K, C), lambda b: (0, 0)),
        ],
        out_specs=[
            pl.BlockSpec((1, C, H * W), lambda b: (b, 0, 0)),
            pl.BlockSpec((1, 8, 128), lambda b: (b, 0, 0)),
        ],
        out_shape=[
            jax.ShapeDtypeStruct((B, C, H * W), jnp.float32),
            jax.ShapeDtypeStruct((B, 8, 128), jnp.int32),
        ],
        compiler_params=pltpu.CompilerParams(
            dimension_semantics=("arbitrary",),
        ),
    )(z, codebook)
    return zq.reshape(B, C, H, W), idx.reshape(-1)


# speedup vs baseline: 1.9503x; 1.0576x over previous
import jax
import jax.numpy as jnp
from jax.experimental import pallas as pl
from jax.experimental.pallas import tpu as pltpu

_B, _C, _H, _W = 16, 64, 32, 32
_HW = _H * _W
_K = 512


_MB = 4


def _vq_body(z_ref, cb_ref, zq_ref, idx_ref):
    cb = cb_ref[...]
    esq = jnp.sum(cb * cb, axis=1, keepdims=True)
    kio = jax.lax.broadcasted_iota(jnp.int32, (_K, _HW), 0)
    for i in range(_MB):
        zb = z_ref[i]
        dot = jax.lax.dot_general(
            cb, zb, (((1,), (0,)), ((), ())),
            preferred_element_type=jnp.float32,
        )
        s = zb * zb
        w = _C
        while w > 1:
            w //= 2
            s = s[:w] + s[w:2 * w]
        zsq = s
        d = zsq - 2.0 * dot + esq
        dmin = jnp.min(d, axis=0, keepdims=True)
        idx = jnp.min(jnp.where(d == dmin, kio, _K), axis=0).astype(jnp.int32)
        oh = (kio == idx[None, :]).astype(jnp.float32)
        zq = jax.lax.dot_general(
            cb, oh, (((0,), (0,)), ((), ())),
            preferred_element_type=jnp.float32,
        )
        zq_ref[i] = zq
        idx_ref[i] = idx.reshape(8, 128)


def kernel(z_e, codebook):
    B, C, H, W = z_e.shape
    z = z_e.reshape(B, C, H * W)
    zq, idx = pl.pallas_call(
        _vq_body,
        grid=(B // _MB,),
        in_specs=[
            pl.BlockSpec((_MB, C, H * W), lambda b: (b, 0, 0)),
            pl.BlockSpec((_K, C), lambda b: (0, 0)),
        ],
        out_specs=[
            pl.BlockSpec((_MB, C, H * W), lambda b: (b, 0, 0)),
            pl.BlockSpec((_MB, 8, 128), lambda b: (b, 0, 0)),
        ],
        out_shape=[
            jax.ShapeDtypeStruct((B, C, H * W), jnp.float32),
            jax.ShapeDtypeStruct((B, 8, 128), jnp.int32),
        ],
        compiler_params=pltpu.CompilerParams(
            dimension_semantics=("arbitrary",),
        ),
    )(z, codebook)
    return zq.reshape(B, C, H, W), idx.reshape(-1)
